# baseline (speedup 1.0000x reference)
import functools

import jax
import jax.numpy as jnp
from jax import lax
from jax.experimental import pallas as pl
from jax.experimental.pallas import tpu as pltpu
from jax.experimental.pallas import tpu_sc as plsc

N = 10000
NP = 1024
E = 320000
D = 128
L = 3

NC = 2
NS = 16
NW = NC * NS
EPW = E // NW
C = 80
NCHUNK = EPW // C
ZR = 16
ZZ = 24
SLAB = (N // NS) // ZR * ZR


def _make_sc_segsum(mode: str):
    mesh = plsc.VectorSubcoreMesh(core_axis_name="c", subcore_axis_name="s")

    out_type = [jax.ShapeDtypeStruct((NC, N, D), jnp.float32)]
    scratch = [
        pltpu.VMEM((NCHUNK, C), jnp.int32),
        pltpu.VMEM((C, D), jnp.float32),
        pltpu.VMEM((ZZ, D), jnp.float32),
        pltpu.VMEM_SHARED((N, D), jnp.float32),
    ]
    if mode == "rows":
        scratch += [
            pltpu.VMEM((EPW,), jnp.int32),
            pltpu.VMEM((C, D), jnp.float32),
            pltpu.SemaphoreType.DMA,
            pltpu.SemaphoreType.DMA,
            pltpu.SemaphoreType.DMA,
            pltpu.SemaphoreType.DMA,
        ]

    def body(*refs):
        if mode == "rows":
            (h_hbm, src_hbm, dst_hbm, s_out, dst_v, rows_a, zrow_v, acc_sh,
             src_v, rows_b, sem_a, sem_b, sem_sa, sem_sb) = refs
        else:
            (dst_hbm, s_out, dst_v, rows_a, zrow_v, acc_sh) = refs

        cid = lax.axis_index("c")
        sid = lax.axis_index("s")
        w = cid * NS + sid

        pltpu.sync_copy(dst_hbm.at[w], dst_v)
        if mode == "rows":
            pltpu.sync_copy(
                src_hbm.at[pl.ds(pl.multiple_of(w * EPW, 8), EPW)], src_v)

        def src_at(j):
            return src_v.at[pl.ds(pl.multiple_of(j * C, 8), C)]

        def zfill(i, _):
            zrow_v[i // 8, pl.ds((i % 8) * 16, 16)] = jnp.zeros((16,), jnp.float32)
            return 0
        lax.fori_loop(0, ZZ * (D // 16), zfill, 0)
        if mode == "deg":
            def ofill(i, _):
                rows_a[i // 8, pl.ds((i % 8) * 16, 16)] = jnp.full(
                    (16,), 1.0, jnp.float32)
                return 0
            lax.fori_loop(0, C * (D // 16), ofill, 0)

        def ztile(k, _):
            r = pl.multiple_of(sid * SLAB + k * ZZ, 8)
            pltpu.sync_copy(zrow_v, acc_sh.at[pl.ds(r, ZZ)])
            return 0
        lax.fori_loop(0, SLAB // ZZ, ztile, 0)
        pltpu.sync_copy(zrow_v.at[pl.ds(0, ZR)], acc_sh.at[pl.ds(NS * SLAB, ZR)])
        plsc.subcore_barrier()

        if mode == "rows":
            def wait_g(buf, sem):
                pltpu.make_async_copy(h_hbm.at[dst_v.at[0]], buf, sem).wait()

            pltpu.async_copy(h_hbm.at[src_at(0)], rows_a, sem_a)

            def step(jj, _):
                b1 = 2 * jj + 1
                b2 = 2 * jj + 2
                gb = pltpu.async_copy(h_hbm.at[src_at(b1)], rows_b, sem_b)
                wait_g(rows_a, sem_a)
                sa = pltpu.async_copy(rows_a, acc_sh.at[dst_v.at[2 * jj]],
                                      sem_sa, add=True)
                sa.wait()
                pltpu.async_copy(h_hbm.at[src_at(b2)], rows_a, sem_a)
                gb.wait()
                sb = pltpu.async_copy(rows_b, acc_sh.at[dst_v.at[b1]],
                                      sem_sb, add=True)
                sb.wait()
                return 0
            lax.fori_loop(0, (NCHUNK - 1) // 2, step, 0)
            wait_g(rows_a, sem_a)
            pltpu.sync_copy(rows_a, acc_sh.at[dst_v.at[NCHUNK - 1]], add=True)
        else:
            def chunk(j, _):
                pltpu.sync_copy(rows_a, acc_sh.at[dst_v.at[j]], add=True)
                return 0
            lax.fori_loop(0, NCHUNK, chunk, 0)
        plsc.subcore_barrier()

        r0 = pl.multiple_of(sid * SLAB, 8)
        pltpu.sync_copy(acc_sh.at[pl.ds(r0, SLAB)],
                        s_out.at[cid, pl.ds(r0, SLAB)])
        pltpu.sync_copy(acc_sh.at[pl.ds(NS * SLAB, ZR)],
                        s_out.at[cid, pl.ds(NS * SLAB, ZR)])

    return pl.kernel(body, out_type=out_type, mesh=mesh,
                     scratch_types=scratch)


_sc_segsum = _make_sc_segsum("rows")
_sc_deg = _make_sc_segsum("deg")


BN = 1000


def _dense_body(relu, h_ref, s0_ref, s1_ref, d0_ref, d1_ref, ws_ref, wn_ref,
                b_ref, g_ref, be_ref, o_ref):
    neigh = s0_ref[0] + s1_ref[0]
    deg16 = d0_ref[0] + d1_ref[0]
    deg = jnp.max(deg16, axis=-1, keepdims=True)
    nm = neigh * (1.0 / jnp.maximum(deg, 1.0))
    y = (jnp.dot(h_ref[...], ws_ref[...], preferred_element_type=jnp.float32)
         + jnp.dot(nm, wn_ref[...], preferred_element_type=jnp.float32)
         + b_ref[...])
    mu = jnp.mean(y, axis=-1, keepdims=True)
    var = jnp.mean((y - mu) ** 2, axis=-1, keepdims=True)
    out = (y - mu) * lax.rsqrt(var + 1e-5) * g_ref[...] + be_ref[...]
    if relu:
        out = jnp.maximum(out, 0.0)
    o_ref[...] = out


def _dense_layer(h, s, dg, ws, wn, b, g, be, relu):
    return pl.pallas_call(
        functools.partial(_dense_body, relu),
        grid=(N // BN,),
        in_specs=[
            pl.BlockSpec((BN, D), lambda i: (i, 0)),
            pl.BlockSpec((1, BN, D), lambda i: (0, i, 0)),
            pl.BlockSpec((1, BN, D), lambda i: (1, i, 0)),
            pl.BlockSpec((1, BN, D), lambda i: (0, i, 0)),
            pl.BlockSpec((1, BN, D), lambda i: (1, i, 0)),
            pl.BlockSpec((D, D), lambda i: (0, 0)),
            pl.BlockSpec((D, D), lambda i: (0, 0)),
            pl.BlockSpec((1, D), lambda i: (0, 0)),
            pl.BlockSpec((1, D), lambda i: (0, 0)),
            pl.BlockSpec((1, D), lambda i: (0, 0)),
        ],
        out_specs=pl.BlockSpec((BN, D), lambda i: (i, 0)),
        out_shape=jax.ShapeDtypeStruct((N, D), jnp.float32),
    )(h, s, s, dg, dg, ws, wn, b, g, be)


def _hp_body(xp_ref, wl_ref, bl_ref, g_ref, be_ref, o_ref):
    hp = xp_ref[...]
    for l in range(L):
        y = (jnp.dot(hp, wl_ref[l], preferred_element_type=jnp.float32)
             + bl_ref[l])
        mu = jnp.mean(y, axis=-1, keepdims=True)
        var = jnp.mean((y - mu) ** 2, axis=-1, keepdims=True)
        hp = (y - mu) * lax.rsqrt(var + 1e-5) * g_ref[l] + be_ref[l]
        if l < L - 1:
            hp = jnp.maximum(hp, 0.0)
    o_ref[...] = hp


def _hp_path(xp, wl, bl, g, be):
    return pl.pallas_call(
        _hp_body,
        out_shape=jax.ShapeDtypeStruct((NP, D), jnp.float32),
    )(xp, wl, bl, g, be)


def kernel(x, xp, edge_index, Ws, Wn, b, Wl, bl, gamma, beta):
    src = edge_index[0]
    dst = edge_index[1].reshape(NW, NCHUNK, C)
    b2 = b.reshape(L, 1, D)
    bl2 = bl.reshape(L, 1, D)
    g2 = gamma.reshape(L, 1, D)
    be2 = beta.reshape(L, 1, D)

    (dg,) = _sc_deg(dst)

    h = x
    for l in range(L):
        (s,) = _sc_segsum(h, src, dst)
        h = _dense_layer(h, s, dg, Ws[l], Wn[l], b2[l], g2[l], be2[l],
                         relu=(l < L - 1))
    hp = _hp_path(xp, Wl, bl2, g2, be2)
    return h, hp


# speedup vs baseline: 1.1411x; 1.1411x over previous
import functools

import jax
import jax.numpy as jnp
from jax import lax
from jax.experimental import pallas as pl
from jax.experimental.pallas import tpu as pltpu
from jax.experimental.pallas import tpu_sc as plsc

N = 10000
NP = 1024
E = 320000
D = 128
L = 3

NC = 2
NS = 16
NW = NC * NS
EPW = E // NW
C = 80
NCHUNK = EPW // C
ZR = 16
ZZ = 24
SLAB = (N // NS) // ZR * ZR
NPAD = 10240
SLABC = NPAD // NS


def _make_sc_segsum():
    mesh = plsc.VectorSubcoreMesh(core_axis_name="c", subcore_axis_name="s")

    out_type = [jax.ShapeDtypeStruct((NC, N, D), jnp.float32)]
    scratch = [
        pltpu.VMEM((NCHUNK, C), jnp.int32),
        pltpu.VMEM((C, D), jnp.float32),
        pltpu.VMEM((ZZ, D), jnp.float32),
        pltpu.VMEM_SHARED((N, D), jnp.float32),
        pltpu.VMEM((EPW,), jnp.int32),
        pltpu.VMEM((C, D), jnp.float32),
        pltpu.SemaphoreType.DMA,
        pltpu.SemaphoreType.DMA,
        pltpu.SemaphoreType.DMA,
        pltpu.SemaphoreType.DMA,
    ]

    def body(*refs):
        (h_hbm, src_hbm, dst_hbm, s_out, dst_v, rows_a, zrow_v, acc_sh,
         src_v, rows_b, sem_a, sem_b, sem_sa, sem_sb) = refs

        cid = lax.axis_index("c")
        sid = lax.axis_index("s")
        w = cid * NS + sid

        pltpu.sync_copy(dst_hbm.at[w], dst_v)
        pltpu.sync_copy(
            src_hbm.at[pl.ds(pl.multiple_of(w * EPW, 8), EPW)], src_v)

        def src_at(j):
            return src_v.at[pl.ds(pl.multiple_of(j * C, 8), C)]

        def zfill(i, _):
            zrow_v[i // 8, pl.ds((i % 8) * 16, 16)] = jnp.zeros((16,), jnp.float32)
            return 0
        lax.fori_loop(0, ZZ * 8, zfill, 0)

        def ztile(k, _):
            r = pl.multiple_of(sid * SLAB + k * ZZ, 8)
            pltpu.sync_copy(zrow_v, acc_sh.at[pl.ds(r, ZZ)])
            return 0
        lax.fori_loop(0, SLAB // ZZ, ztile, 0)
        pltpu.sync_copy(zrow_v.at[pl.ds(0, ZR)], acc_sh.at[pl.ds(NS * SLAB, ZR)])
        plsc.subcore_barrier()

        def wait_g(buf, sem):
            pltpu.make_async_copy(h_hbm.at[dst_v.at[0]], buf, sem).wait()

        pltpu.async_copy(h_hbm.at[src_at(0)], rows_a, sem_a)

        def step(jj, _):
            b1 = 2 * jj + 1
            b2 = 2 * jj + 2
            gb = pltpu.async_copy(h_hbm.at[src_at(b1)], rows_b, sem_b)
            wait_g(rows_a, sem_a)
            sa = pltpu.async_copy(rows_a, acc_sh.at[dst_v.at[2 * jj]],
                                  sem_sa, add=True)
            sa.wait()
            pltpu.async_copy(h_hbm.at[src_at(b2)], rows_a, sem_a)
            gb.wait()
            sb = pltpu.async_copy(rows_b, acc_sh.at[dst_v.at[b1]],
                                  sem_sb, add=True)
            sb.wait()
            return 0
        lax.fori_loop(0, (NCHUNK - 1) // 2, step, 0)
        wait_g(rows_a, sem_a)
        pltpu.sync_copy(rows_a, acc_sh.at[dst_v.at[NCHUNK - 1]], add=True)
        plsc.subcore_barrier()

        r0 = pl.multiple_of(sid * SLAB, 8)
        pltpu.sync_copy(acc_sh.at[pl.ds(r0, SLAB)],
                        s_out.at[cid, pl.ds(r0, SLAB)])
        pltpu.sync_copy(acc_sh.at[pl.ds(NS * SLAB, ZR)],
                        s_out.at[cid, pl.ds(NS * SLAB, ZR)])

    return pl.kernel(body, out_type=out_type, mesh=mesh,
                     scratch_types=scratch)


def _make_sc_deg():
    mesh = plsc.VectorSubcoreMesh(core_axis_name="c", subcore_axis_name="s")

    out_type = [jax.ShapeDtypeStruct((NC, 8, NPAD), jnp.float32)]
    scratch = [
        pltpu.VMEM((NCHUNK, C), jnp.int32),
        pltpu.VMEM((NPAD,), jnp.float32),
        pltpu.VMEM((NS, SLABC), jnp.float32),
        pltpu.VMEM((SLABC,), jnp.float32),
        pltpu.VMEM_SHARED((NS, NPAD), jnp.float32),
    ]

    def body(dst_hbm, out, dst_v, hist_v, slab_v, outv_v, hsh):
        cid = lax.axis_index("c")
        sid = lax.axis_index("s")
        w = cid * NS + sid
        pltpu.sync_copy(dst_hbm.at[w], dst_v)

        zero16 = jnp.zeros((16,), jnp.float32)
        one16 = jnp.full((16,), 1.0, jnp.float32)

        def zh(i, _):
            hist_v[pl.ds(i * 16, 16)] = zero16
            return 0
        lax.fori_loop(0, NPAD // 16, zh, 0)

        def hchunk(j, _):
            for q in range(C // 16):
                idx = dst_v[j, pl.ds(q * 16, 16)]
                plsc.addupdate_scatter(hist_v, [idx], one16)
            return 0
        lax.fori_loop(0, NCHUNK, hchunk, 0)

        pltpu.sync_copy(hist_v, hsh.at[sid])
        plsc.subcore_barrier()

        c0 = pl.multiple_of(sid * SLABC, 128)
        pltpu.sync_copy(hsh.at[:, pl.ds(c0, SLABC)], slab_v)

        def red(q, _):
            acc = zero16
            for r in range(NS):
                acc = acc + slab_v[r, pl.ds(q * 16, 16)]
            outv_v[pl.ds(q * 16, 16)] = acc
            return 0
        lax.fori_loop(0, SLABC // 16, red, 0)

        pltpu.sync_copy(outv_v, out.at[cid, 0, pl.ds(c0, SLABC)])

    return pl.kernel(
        body, out_type=out_type, mesh=mesh, scratch_types=scratch,
        compiler_params=pltpu.CompilerParams(needs_layout_passes=False))


_sc_segsum = _make_sc_segsum()
_sc_deg = _make_sc_deg()


BN = 1000


def _dense_body(relu, h_ref, s0_ref, s1_ref, d_ref, ws_ref, wn_ref,
                b_ref, g_ref, be_ref, o_ref):
    neigh = s0_ref[0] + s1_ref[0]
    deg = d_ref[...]
    nm = neigh * (1.0 / jnp.maximum(deg, 1.0))
    y = (jnp.dot(h_ref[...], ws_ref[...], preferred_element_type=jnp.float32)
         + jnp.dot(nm, wn_ref[...], preferred_element_type=jnp.float32)
         + b_ref[...])
    mu = jnp.mean(y, axis=-1, keepdims=True)
    var = jnp.mean((y - mu) ** 2, axis=-1, keepdims=True)
    out = (y - mu) * lax.rsqrt(var + 1e-5) * g_ref[...] + be_ref[...]
    if relu:
        out = jnp.maximum(out, 0.0)
    o_ref[...] = out


def _dense_layer(h, s, dg, ws, wn, b, g, be, relu):
    return pl.pallas_call(
        functools.partial(_dense_body, relu),
        grid=(N // BN,),
        in_specs=[
            pl.BlockSpec((BN, D), lambda i: (i, 0)),
            pl.BlockSpec((1, BN, D), lambda i: (0, i, 0)),
            pl.BlockSpec((1, BN, D), lambda i: (1, i, 0)),
            pl.BlockSpec((BN, 1), lambda i: (i, 0)),
            pl.BlockSpec((D, D), lambda i: (0, 0)),
            pl.BlockSpec((D, D), lambda i: (0, 0)),
            pl.BlockSpec((1, D), lambda i: (0, 0)),
            pl.BlockSpec((1, D), lambda i: (0, 0)),
            pl.BlockSpec((1, D), lambda i: (0, 0)),
        ],
        out_specs=pl.BlockSpec((BN, D), lambda i: (i, 0)),
        out_shape=jax.ShapeDtypeStruct((N, D), jnp.float32),
    )(h, s, s, dg, ws, wn, b, g, be)


def _hp_body(xp_ref, wl_ref, bl_ref, g_ref, be_ref, o_ref):
    hp = xp_ref[...]
    for l in range(L):
        y = (jnp.dot(hp, wl_ref[l], preferred_element_type=jnp.float32)
             + bl_ref[l])
        mu = jnp.mean(y, axis=-1, keepdims=True)
        var = jnp.mean((y - mu) ** 2, axis=-1, keepdims=True)
        hp = (y - mu) * lax.rsqrt(var + 1e-5) * g_ref[l] + be_ref[l]
        if l < L - 1:
            hp = jnp.maximum(hp, 0.0)
    o_ref[...] = hp


def _hp_path(xp, wl, bl, g, be):
    return pl.pallas_call(
        _hp_body,
        out_shape=jax.ShapeDtypeStruct((NP, D), jnp.float32),
    )(xp, wl, bl, g, be)


def kernel(x, xp, edge_index, Ws, Wn, b, Wl, bl, gamma, beta):
    src = edge_index[0]
    dst = edge_index[1].reshape(NW, NCHUNK, C)
    b2 = b.reshape(L, 1, D)
    bl2 = bl.reshape(L, 1, D)
    g2 = gamma.reshape(L, 1, D)
    be2 = beta.reshape(L, 1, D)

    (dg3,) = _sc_deg(dst)
    dg = (dg3[0, 0, :N] + dg3[1, 0, :N]).reshape(N, 1)

    h = x
    for l in range(L):
        (s,) = _sc_segsum(h, src, dst)
        h = _dense_layer(h, s, dg, Ws[l], Wn[l], b2[l], g2[l], be2[l],
                         relu=(l < L - 1))
    hp = _hp_path(xp, Wl, bl2, g2, be2)
    return h, hp


# speedup vs baseline: 1.1952x; 1.0474x over previous
import functools

import jax
import jax.numpy as jnp
from jax import lax
from jax.experimental import pallas as pl
from jax.experimental.pallas import tpu as pltpu
from jax.experimental.pallas import tpu_sc as plsc

N = 10000
NP = 1024
E = 320000
D = 128
L = 3

NC = 2
NS = 16
NW = NC * NS
EPW = E // NW
C = 104
NCHUNK = EPW // C
TAIL = EPW - NCHUNK * C
ZR = 16
ZZ = 24
SLAB = (N // NS) // ZR * ZR
NPAD = 10240
SLABC = NPAD // NS


def _make_sc_segsum():
    mesh = plsc.VectorSubcoreMesh(core_axis_name="c", subcore_axis_name="s")

    out_type = [jax.ShapeDtypeStruct((NC, N, D), jnp.float32)]
    scratch = [
        pltpu.VMEM((NCHUNK, C), jnp.int32),
        pltpu.VMEM((TAIL,), jnp.int32),
        pltpu.VMEM((C, D), jnp.float32),
        pltpu.VMEM_SHARED((N, D), jnp.float32),
        pltpu.VMEM((EPW,), jnp.int32),
        pltpu.VMEM((C, D), jnp.float32),
        pltpu.SemaphoreType.DMA,
        pltpu.SemaphoreType.DMA,
        pltpu.SemaphoreType.DMA,
        pltpu.SemaphoreType.DMA,
    ]

    def body(*refs):
        (h_hbm, src_hbm, dstm_hbm, dstt_hbm, s_out, dst_v, dstt_v, rows_a,
         acc_sh, src_v, rows_b, sem_a, sem_b, sem_sa, sem_sb) = refs

        cid = lax.axis_index("c")
        sid = lax.axis_index("s")
        w = cid * NS + sid

        pltpu.sync_copy(dstm_hbm.at[w], dst_v)
        pltpu.sync_copy(dstt_hbm.at[w], dstt_v)
        pltpu.sync_copy(
            src_hbm.at[pl.ds(pl.multiple_of(w * EPW, 8), EPW)], src_v)

        def src_at(j, n=C):
            return src_v.at[pl.ds(pl.multiple_of(j * C, 8), n)]

        def zfill(i, _):
            rows_b[i // 8, pl.ds((i % 8) * 16, 16)] = jnp.zeros((16,), jnp.float32)
            return 0
        lax.fori_loop(0, C * 8, zfill, 0)

        for k in range(SLAB // C):
            r = sid * SLAB + k * C
            pltpu.sync_copy(rows_b, acc_sh.at[pl.ds(r, C)])
        pltpu.sync_copy(rows_b.at[pl.ds(0, ZR)], acc_sh.at[pl.ds(NS * SLAB, ZR)])
        plsc.subcore_barrier()

        def wait_g(buf, sem):
            pltpu.make_async_copy(h_hbm.at[dst_v.at[0]], buf, sem).wait()

        pltpu.async_copy(h_hbm.at[src_at(0)], rows_a, sem_a)

        def step(jj, _):
            b1 = 2 * jj + 1
            b2 = 2 * jj + 2
            gb = pltpu.async_copy(h_hbm.at[src_at(b1)], rows_b, sem_b)
            wait_g(rows_a, sem_a)
            sa = pltpu.async_copy(rows_a, acc_sh.at[dst_v.at[2 * jj]],
                                  sem_sa, add=True)
            sa.wait()
            pltpu.async_copy(h_hbm.at[src_at(b2)], rows_a, sem_a)
            gb.wait()
            sb = pltpu.async_copy(rows_b, acc_sh.at[dst_v.at[b1]],
                                  sem_sb, add=True)
            sb.wait()
            return 0
        lax.fori_loop(0, (NCHUNK - 2) // 2, step, 0)
        g95 = pltpu.async_copy(h_hbm.at[src_at(NCHUNK - 1)], rows_b, sem_b)
        wait_g(rows_a, sem_a)
        pltpu.sync_copy(rows_a, acc_sh.at[dst_v.at[NCHUNK - 2]], add=True)
        gt = pltpu.async_copy(h_hbm.at[src_at(NCHUNK, TAIL)],
                              rows_a.at[pl.ds(0, TAIL)], sem_a)
        g95.wait()
        pltpu.sync_copy(rows_b, acc_sh.at[dst_v.at[NCHUNK - 1]], add=True)
        gt.wait()
        pltpu.sync_copy(rows_a.at[pl.ds(0, TAIL)], acc_sh.at[dstt_v],
                        add=True)
        plsc.subcore_barrier()

        r0 = pl.multiple_of(sid * SLAB, 8)
        pltpu.sync_copy(acc_sh.at[pl.ds(r0, SLAB)],
                        s_out.at[cid, pl.ds(r0, SLAB)])
        pltpu.sync_copy(acc_sh.at[pl.ds(NS * SLAB, ZR)],
                        s_out.at[cid, pl.ds(NS * SLAB, ZR)])

    return pl.kernel(body, out_type=out_type, mesh=mesh,
                     scratch_types=scratch)


def _make_sc_deg():
    mesh = plsc.VectorSubcoreMesh(core_axis_name="c", subcore_axis_name="s")

    out_type = [jax.ShapeDtypeStruct((NC, 8, NPAD), jnp.float32)]
    scratch = [
        pltpu.VMEM((EPW,), jnp.int32),
        pltpu.VMEM((NPAD,), jnp.float32),
        pltpu.VMEM((NS, SLABC), jnp.float32),
        pltpu.VMEM((SLABC,), jnp.float32),
        pltpu.VMEM_SHARED((NS, NPAD), jnp.float32),
    ]

    def body(dst_hbm, out, dst_v, hist_v, slab_v, outv_v, hsh):
        cid = lax.axis_index("c")
        sid = lax.axis_index("s")
        w = cid * NS + sid
        pltpu.sync_copy(dst_hbm.at[w], dst_v)

        zero16 = jnp.zeros((16,), jnp.float32)
        one16 = jnp.full((16,), 1.0, jnp.float32)

        def zh(i, _):
            hist_v[pl.ds(i * 16, 16)] = zero16
            return 0
        lax.fori_loop(0, NPAD // 16, zh, 0)

        def hchunk(i, _):
            idx = dst_v[pl.ds(i * 16, 16)]
            plsc.addupdate_scatter(hist_v, [idx], one16)
            return 0
        lax.fori_loop(0, EPW // 16, hchunk, 0)

        pltpu.sync_copy(hist_v, hsh.at[sid])
        plsc.subcore_barrier()

        c0 = pl.multiple_of(sid * SLABC, 128)
        pltpu.sync_copy(hsh.at[:, pl.ds(c0, SLABC)], slab_v)

        def red(q, _):
            acc = zero16
            for r in range(NS):
                acc = acc + slab_v[r, pl.ds(q * 16, 16)]
            outv_v[pl.ds(q * 16, 16)] = acc
            return 0
        lax.fori_loop(0, SLABC // 16, red, 0)

        pltpu.sync_copy(outv_v, out.at[cid, 0, pl.ds(c0, SLABC)])

    return pl.kernel(
        body, out_type=out_type, mesh=mesh, scratch_types=scratch,
        compiler_params=pltpu.CompilerParams(needs_layout_passes=False))


_sc_segsum = _make_sc_segsum()
_sc_deg = _make_sc_deg()


BN = 1000


def _dense_body(relu, h_ref, s0_ref, s1_ref, d_ref, ws_ref, wn_ref,
                b_ref, g_ref, be_ref, o_ref):
    neigh = s0_ref[0] + s1_ref[0]
    deg = d_ref[...]
    nm = neigh * (1.0 / jnp.maximum(deg, 1.0))
    y = (jnp.dot(h_ref[...], ws_ref[...], preferred_element_type=jnp.float32)
         + jnp.dot(nm, wn_ref[...], preferred_element_type=jnp.float32)
         + b_ref[...])
    mu = jnp.mean(y, axis=-1, keepdims=True)
    var = jnp.mean((y - mu) ** 2, axis=-1, keepdims=True)
    out = (y - mu) * lax.rsqrt(var + 1e-5) * g_ref[...] + be_ref[...]
    if relu:
        out = jnp.maximum(out, 0.0)
    o_ref[...] = out


def _dense_layer(h, s, dg, ws, wn, b, g, be, relu):
    return pl.pallas_call(
        functools.partial(_dense_body, relu),
        grid=(N // BN,),
        in_specs=[
            pl.BlockSpec((BN, D), lambda i: (i, 0)),
            pl.BlockSpec((1, BN, D), lambda i: (0, i, 0)),
            pl.BlockSpec((1, BN, D), lambda i: (1, i, 0)),
            pl.BlockSpec((BN, 1), lambda i: (i, 0)),
            pl.BlockSpec((D, D), lambda i: (0, 0)),
            pl.BlockSpec((D, D), lambda i: (0, 0)),
            pl.BlockSpec((1, D), lambda i: (0, 0)),
            pl.BlockSpec((1, D), lambda i: (0, 0)),
            pl.BlockSpec((1, D), lambda i: (0, 0)),
        ],
        out_specs=pl.BlockSpec((BN, D), lambda i: (i, 0)),
        out_shape=jax.ShapeDtypeStruct((N, D), jnp.float32),
    )(h, s, s, dg, ws, wn, b, g, be)


def _hp_body(xp_ref, wl_ref, bl_ref, g_ref, be_ref, o_ref):
    hp = xp_ref[...]
    for l in range(L):
        y = (jnp.dot(hp, wl_ref[l], preferred_element_type=jnp.float32)
             + bl_ref[l])
        mu = jnp.mean(y, axis=-1, keepdims=True)
        var = jnp.mean((y - mu) ** 2, axis=-1, keepdims=True)
        hp = (y - mu) * lax.rsqrt(var + 1e-5) * g_ref[l] + be_ref[l]
        if l < L - 1:
            hp = jnp.maximum(hp, 0.0)
    o_ref[...] = hp


def _hp_path(xp, wl, bl, g, be):
    return pl.pallas_call(
        _hp_body,
        out_shape=jax.ShapeDtypeStruct((NP, D), jnp.float32),
    )(xp, wl, bl, g, be)


def kernel(x, xp, edge_index, Ws, Wn, b, Wl, bl, gamma, beta):
    src = edge_index[0]
    dst2 = edge_index[1].reshape(NW, EPW)
    dstm = dst2[:, :NCHUNK * C].reshape(NW, NCHUNK, C)
    dstt = dst2[:, NCHUNK * C:]
    b2 = b.reshape(L, 1, D)
    bl2 = bl.reshape(L, 1, D)
    g2 = gamma.reshape(L, 1, D)
    be2 = beta.reshape(L, 1, D)

    (dg3,) = _sc_deg(dst2)
    dg = (dg3[0, 0, :N] + dg3[1, 0, :N]).reshape(N, 1)

    h = x
    for l in range(L):
        (s,) = _sc_segsum(h, src, dstm, dstt)
        h = _dense_layer(h, s, dg, Ws[l], Wn[l], b2[l], g2[l], be2[l],
                         relu=(l < L - 1))
    hp = _hp_path(xp, Wl, bl2, g2, be2)
    return h, hp


# speedup vs baseline: 1.2140x; 1.0157x over previous
import functools

import jax
import jax.numpy as jnp
from jax import lax
from jax.experimental import pallas as pl
from jax.experimental.pallas import tpu as pltpu
from jax.experimental.pallas import tpu_sc as plsc

N = 10000
NP = 1024
E = 320000
D = 128
L = 3

NC = 2
NS = 16
NW = NC * NS
EPW = E // NW
C = 104
NCHUNK = EPW // C
TAIL = EPW - NCHUNK * C
ZR = 16
ZZ = 24
SLAB = (N // NS) // ZR * ZR
NPAD = 10240
SLABC = NPAD // NS


def _make_sc_segsum():
    mesh = plsc.VectorSubcoreMesh(core_axis_name="c", subcore_axis_name="s")

    out_type = [jax.ShapeDtypeStruct((NC, N, D), jnp.float32)]
    scratch = [
        pltpu.VMEM((NCHUNK, C), jnp.int32),
        pltpu.VMEM((TAIL,), jnp.int32),
        pltpu.VMEM((C, D), jnp.float32),
        pltpu.VMEM_SHARED((N, D), jnp.float32),
        pltpu.VMEM((EPW,), jnp.int32),
        pltpu.VMEM((C, D), jnp.float32),
        pltpu.SemaphoreType.DMA,
        pltpu.SemaphoreType.DMA,
        pltpu.SemaphoreType.DMA,
        pltpu.SemaphoreType.DMA,
    ]

    def body(*refs):
        (h_hbm, src_hbm, dstm_hbm, dstt_hbm, s_out, dst_v, dstt_v, rows_a,
         acc_sh, src_v, rows_b, sem_a, sem_b, sem_sa, sem_sb) = refs

        cid = lax.axis_index("c")
        sid = lax.axis_index("s")
        w = cid * NS + sid

        pltpu.sync_copy(dstm_hbm.at[w], dst_v)
        pltpu.sync_copy(dstt_hbm.at[w], dstt_v)
        pltpu.sync_copy(
            src_hbm.at[pl.ds(pl.multiple_of(w * EPW, 8), EPW)], src_v)

        def src_at(j, n=C):
            return src_v.at[pl.ds(pl.multiple_of(j * C, 8), n)]

        def zfill(i, _):
            rows_b[i // 8, pl.ds((i % 8) * 16, 16)] = jnp.zeros((16,), jnp.float32)
            return 0
        lax.fori_loop(0, C * 8, zfill, 0)

        for k in range(SLAB // C):
            r = sid * SLAB + k * C
            pltpu.sync_copy(rows_b, acc_sh.at[pl.ds(r, C)])
        pltpu.sync_copy(rows_b.at[pl.ds(0, ZR)], acc_sh.at[pl.ds(NS * SLAB, ZR)])
        plsc.subcore_barrier()

        def wait_g(buf, sem):
            pltpu.make_async_copy(h_hbm.at[dst_v.at[0]], buf, sem).wait()

        pltpu.async_copy(h_hbm.at[src_at(0)], rows_a, sem_a)

        def step(jj, _):
            b1 = 2 * jj + 1
            b2 = 2 * jj + 2
            gb = pltpu.async_copy(h_hbm.at[src_at(b1)], rows_b, sem_b)
            wait_g(rows_a, sem_a)
            sa = pltpu.async_copy(rows_a, acc_sh.at[dst_v.at[2 * jj]],
                                  sem_sa, add=True)
            sa.wait()
            pltpu.async_copy(h_hbm.at[src_at(b2)], rows_a, sem_a)
            gb.wait()
            sb = pltpu.async_copy(rows_b, acc_sh.at[dst_v.at[b1]],
                                  sem_sb, add=True)
            sb.wait()
            return 0
        lax.fori_loop(0, (NCHUNK - 2) // 2, step, 0)
        g95 = pltpu.async_copy(h_hbm.at[src_at(NCHUNK - 1)], rows_b, sem_b)
        wait_g(rows_a, sem_a)
        pltpu.sync_copy(rows_a, acc_sh.at[dst_v.at[NCHUNK - 2]], add=True)
        gt = pltpu.async_copy(h_hbm.at[src_at(NCHUNK, TAIL)],
                              rows_a.at[pl.ds(0, TAIL)], sem_a)
        g95.wait()
        pltpu.sync_copy(rows_b, acc_sh.at[dst_v.at[NCHUNK - 1]], add=True)
        gt.wait()
        pltpu.sync_copy(rows_a.at[pl.ds(0, TAIL)], acc_sh.at[dstt_v],
                        add=True)
        plsc.subcore_barrier()

        r0 = pl.multiple_of(sid * SLAB, 8)
        pltpu.sync_copy(acc_sh.at[pl.ds(r0, SLAB)],
                        s_out.at[cid, pl.ds(r0, SLAB)])
        pltpu.sync_copy(acc_sh.at[pl.ds(NS * SLAB, ZR)],
                        s_out.at[cid, pl.ds(NS * SLAB, ZR)])

    return pl.kernel(body, out_type=out_type, mesh=mesh,
                     scratch_types=scratch)


def _make_sc_deg():
    mesh = plsc.VectorSubcoreMesh(core_axis_name="c", subcore_axis_name="s")

    out_type = [jax.ShapeDtypeStruct((NC, 8, NPAD), jnp.float32)]
    scratch = [
        pltpu.VMEM((EPW,), jnp.int32),
        pltpu.VMEM((NPAD,), jnp.float32),
        pltpu.VMEM((NS, SLABC), jnp.float32),
        pltpu.VMEM((SLABC,), jnp.float32),
        pltpu.VMEM_SHARED((NS, NPAD), jnp.float32),
    ]

    def body(dst_hbm, out, dst_v, hist_v, slab_v, outv_v, hsh):
        cid = lax.axis_index("c")
        sid = lax.axis_index("s")
        w = cid * NS + sid
        pltpu.sync_copy(dst_hbm.at[w], dst_v)

        zero16 = jnp.zeros((16,), jnp.float32)
        one16 = jnp.full((16,), 1.0, jnp.float32)

        def zh(i, _):
            hist_v[pl.ds(i * 16, 16)] = zero16
            return 0
        lax.fori_loop(0, NPAD // 16, zh, 0)

        def hchunk(i, _):
            idx = dst_v[pl.ds(i * 16, 16)]
            plsc.addupdate_scatter(hist_v, [idx], one16)
            return 0
        lax.fori_loop(0, EPW // 16, hchunk, 0)

        pltpu.sync_copy(hist_v, hsh.at[sid])
        plsc.subcore_barrier()

        c0 = pl.multiple_of(sid * SLABC, 128)
        pltpu.sync_copy(hsh.at[:, pl.ds(c0, SLABC)], slab_v)

        def red(q, _):
            acc = zero16
            for r in range(NS):
                acc = acc + slab_v[r, pl.ds(q * 16, 16)]
            outv_v[pl.ds(q * 16, 16)] = acc
            return 0
        lax.fori_loop(0, SLABC // 16, red, 0)

        pltpu.sync_copy(outv_v, out.at[cid, 0, pl.ds(c0, SLABC)])

    return pl.kernel(
        body, out_type=out_type, mesh=mesh, scratch_types=scratch,
        compiler_params=pltpu.CompilerParams(needs_layout_passes=False))


_sc_segsum = _make_sc_segsum()
_sc_deg = _make_sc_deg()


BN = 2000


def _dense_body(relu, h_ref, s0_ref, s1_ref, d_ref, ws_ref, wn_ref,
                b_ref, g_ref, be_ref, o_ref):
    neigh = s0_ref[0] + s1_ref[0]
    deg = d_ref[...]
    nm = neigh * (1.0 / jnp.maximum(deg, 1.0))
    y = (jnp.dot(h_ref[...], ws_ref[...], preferred_element_type=jnp.float32)
         + jnp.dot(nm, wn_ref[...], preferred_element_type=jnp.float32)
         + b_ref[...])
    mu = jnp.mean(y, axis=-1, keepdims=True)
    var = jnp.mean((y - mu) ** 2, axis=-1, keepdims=True)
    out = (y - mu) * lax.rsqrt(var + 1e-5) * g_ref[...] + be_ref[...]
    if relu:
        out = jnp.maximum(out, 0.0)
    o_ref[...] = out


def _dense_layer(h, s, dg, ws, wn, b, g, be, relu):
    return pl.pallas_call(
        functools.partial(_dense_body, relu),
        grid=(N // BN,),
        in_specs=[
            pl.BlockSpec((BN, D), lambda i: (i, 0)),
            pl.BlockSpec((1, BN, D), lambda i: (0, i, 0)),
            pl.BlockSpec((1, BN, D), lambda i: (1, i, 0)),
            pl.BlockSpec((BN, 1), lambda i: (i, 0)),
            pl.BlockSpec((D, D), lambda i: (0, 0)),
            pl.BlockSpec((D, D), lambda i: (0, 0)),
            pl.BlockSpec((1, D), lambda i: (0, 0)),
            pl.BlockSpec((1, D), lambda i: (0, 0)),
            pl.BlockSpec((1, D), lambda i: (0, 0)),
        ],
        out_specs=pl.BlockSpec((BN, D), lambda i: (i, 0)),
        out_shape=jax.ShapeDtypeStruct((N, D), jnp.float32),
    )(h, s, s, dg, ws, wn, b, g, be)


def _hp_body(xp_ref, wl_ref, bl_ref, g_ref, be_ref, o_ref):
    hp = xp_ref[...]
    for l in range(L):
        y = (jnp.dot(hp, wl_ref[l], preferred_element_type=jnp.float32)
             + bl_ref[l])
        mu = jnp.mean(y, axis=-1, keepdims=True)
        var = jnp.mean((y - mu) ** 2, axis=-1, keepdims=True)
        hp = (y - mu) * lax.rsqrt(var + 1e-5) * g_ref[l] + be_ref[l]
        if l < L - 1:
            hp = jnp.maximum(hp, 0.0)
    o_ref[...] = hp


def _hp_path(xp, wl, bl, g, be):
    return pl.pallas_call(
        _hp_body,
        out_shape=jax.ShapeDtypeStruct((NP, D), jnp.float32),
    )(xp, wl, bl, g, be)


def kernel(x, xp, edge_index, Ws, Wn, b, Wl, bl, gamma, beta):
    src = edge_index[0]
    dst2 = edge_index[1].reshape(NW, EPW)
    dstm = dst2[:, :NCHUNK * C].reshape(NW, NCHUNK, C)
    dstt = dst2[:, NCHUNK * C:]
    b2 = b.reshape(L, 1, D)
    bl2 = bl.reshape(L, 1, D)
    g2 = gamma.reshape(L, 1, D)
    be2 = beta.reshape(L, 1, D)

    (dg3,) = _sc_deg(dst2)
    dg = (dg3[0, 0, :N] + dg3[1, 0, :N]).reshape(N, 1)

    h = x
    for l in range(L):
        (s,) = _sc_segsum(h, src, dstm, dstt)
        h = _dense_layer(h, s, dg, Ws[l], Wn[l], b2[l], g2[l], be2[l],
                         relu=(l < L - 1))
    hp = _hp_path(xp, Wl, bl2, g2, be2)
    return h, hp


# speedup vs baseline: 1.2202x; 1.0051x over previous
import functools

import jax
import jax.numpy as jnp
from jax import lax
from jax.experimental import pallas as pl
from jax.experimental.pallas import tpu as pltpu
from jax.experimental.pallas import tpu_sc as plsc

N = 10000
NP = 1024
E = 320000
D = 128
L = 3

NC = 2
NS = 16
NW = NC * NS
EPW = E // NW
C = 104
NCHUNK = EPW // C
TAIL = EPW - NCHUNK * C
ZR = 16
ZZ = 24
SLAB = (N // NS) // ZR * ZR
NPAD = 10240
SLABC = NPAD // NS


def _make_sc_segsum():
    mesh = plsc.VectorSubcoreMesh(core_axis_name="c", subcore_axis_name="s")

    out_type = [jax.ShapeDtypeStruct((NC, N, D), jnp.float32)]
    scratch = [
        pltpu.VMEM((NCHUNK, C), jnp.int32),
        pltpu.VMEM((TAIL,), jnp.int32),
        pltpu.VMEM((C, D), jnp.float32),
        pltpu.VMEM_SHARED((N, D), jnp.float32),
        pltpu.VMEM((EPW,), jnp.int32),
        pltpu.VMEM((C, D), jnp.float32),
        pltpu.SemaphoreType.DMA,
        pltpu.SemaphoreType.DMA,
        pltpu.SemaphoreType.DMA,
        pltpu.SemaphoreType.DMA,
    ]

    def body(*refs):
        (h_hbm, src_hbm, dstm_hbm, dstt_hbm, s_out, dst_v, dstt_v, rows_a,
         acc_sh, src_v, rows_b, sem_a, sem_b, sem_sa, sem_sb) = refs

        cid = lax.axis_index("c")
        sid = lax.axis_index("s")
        w = cid * NS + sid

        pltpu.sync_copy(dstm_hbm.at[w], dst_v)
        pltpu.sync_copy(dstt_hbm.at[w], dstt_v)
        pltpu.sync_copy(
            src_hbm.at[pl.ds(pl.multiple_of(w * EPW, 8), EPW)], src_v)

        def src_at(j, n=C):
            return src_v.at[pl.ds(pl.multiple_of(j * C, 8), n)]

        def zfill(i, _):
            rows_b[i // 8, pl.ds((i % 8) * 16, 16)] = jnp.zeros((16,), jnp.float32)
            return 0
        lax.fori_loop(0, C * 8, zfill, 0)

        for k in range(SLAB // C):
            r = sid * SLAB + k * C
            pltpu.sync_copy(rows_b, acc_sh.at[pl.ds(r, C)])
        pltpu.sync_copy(rows_b.at[pl.ds(0, ZR)], acc_sh.at[pl.ds(NS * SLAB, ZR)])
        plsc.subcore_barrier()

        def wait_g(buf, sem):
            pltpu.make_async_copy(h_hbm.at[dst_v.at[0]], buf, sem).wait()

        pltpu.async_copy(h_hbm.at[src_at(0)], rows_a, sem_a)

        def step(jj, _):
            b1 = 2 * jj + 1
            b2 = 2 * jj + 2
            gb = pltpu.async_copy(h_hbm.at[src_at(b1)], rows_b, sem_b)
            wait_g(rows_a, sem_a)
            sa = pltpu.async_copy(rows_a, acc_sh.at[dst_v.at[2 * jj]],
                                  sem_sa, add=True)
            sa.wait()
            pltpu.async_copy(h_hbm.at[src_at(b2)], rows_a, sem_a)
            gb.wait()
            sb = pltpu.async_copy(rows_b, acc_sh.at[dst_v.at[b1]],
                                  sem_sb, add=True)
            sb.wait()
            return 0
        lax.fori_loop(0, (NCHUNK - 2) // 2, step, 0)
        g95 = pltpu.async_copy(h_hbm.at[src_at(NCHUNK - 1)], rows_b, sem_b)
        wait_g(rows_a, sem_a)
        pltpu.sync_copy(rows_a, acc_sh.at[dst_v.at[NCHUNK - 2]], add=True)
        gt = pltpu.async_copy(h_hbm.at[src_at(NCHUNK, TAIL)],
                              rows_a.at[pl.ds(0, TAIL)], sem_a)
        g95.wait()
        pltpu.sync_copy(rows_b, acc_sh.at[dst_v.at[NCHUNK - 1]], add=True)
        gt.wait()
        pltpu.sync_copy(rows_a.at[pl.ds(0, TAIL)], acc_sh.at[dstt_v],
                        add=True)
        plsc.subcore_barrier()

        r0 = pl.multiple_of(sid * SLAB, 8)
        pltpu.sync_copy(acc_sh.at[pl.ds(r0, SLAB)],
                        s_out.at[cid, pl.ds(r0, SLAB)])
        pltpu.sync_copy(acc_sh.at[pl.ds(NS * SLAB, ZR)],
                        s_out.at[cid, pl.ds(NS * SLAB, ZR)])

    return pl.kernel(body, out_type=out_type, mesh=mesh,
                     scratch_types=scratch)


def _make_sc_deg():
    mesh = plsc.VectorSubcoreMesh(core_axis_name="c", subcore_axis_name="s")

    out_type = [jax.ShapeDtypeStruct((NC, 8, NPAD), jnp.float32)]
    scratch = [
        pltpu.VMEM((EPW,), jnp.int32),
        pltpu.VMEM((NPAD,), jnp.float32),
        pltpu.VMEM((NS, SLABC), jnp.float32),
        pltpu.VMEM((SLABC,), jnp.float32),
        pltpu.VMEM_SHARED((NS, NPAD), jnp.float32),
    ]

    def body(dst_hbm, out, dst_v, hist_v, slab_v, outv_v, hsh):
        cid = lax.axis_index("c")
        sid = lax.axis_index("s")
        w = cid * NS + sid
        pltpu.sync_copy(
            dst_hbm.at[pl.ds(pl.multiple_of(w * EPW, 8), EPW)], dst_v)

        zero16 = jnp.zeros((16,), jnp.float32)
        one16 = jnp.full((16,), 1.0, jnp.float32)

        def zh(i, _):
            hist_v[pl.ds(i * 16, 16)] = zero16
            return 0
        lax.fori_loop(0, NPAD // 16, zh, 0)

        def hchunk(i, _):
            idx = dst_v[pl.ds(i * 16, 16)]
            plsc.addupdate_scatter(hist_v, [idx], one16)
            return 0
        lax.fori_loop(0, EPW // 16, hchunk, 0)

        pltpu.sync_copy(hist_v, hsh.at[sid])
        plsc.subcore_barrier()

        c0 = pl.multiple_of(sid * SLABC, 128)
        pltpu.sync_copy(hsh.at[:, pl.ds(c0, SLABC)], slab_v)

        def red(q, _):
            acc = zero16
            for r in range(NS):
                acc = acc + slab_v[r, pl.ds(q * 16, 16)]
            outv_v[pl.ds(q * 16, 16)] = acc
            return 0
        lax.fori_loop(0, SLABC // 16, red, 0)

        pltpu.sync_copy(outv_v, out.at[cid, 0, pl.ds(c0, SLABC)])

    return pl.kernel(
        body, out_type=out_type, mesh=mesh, scratch_types=scratch,
        compiler_params=pltpu.CompilerParams(needs_layout_passes=False))


_sc_segsum = _make_sc_segsum()
_sc_deg = _make_sc_deg()


BN = 2000


def _dense_body(relu, h_ref, s0_ref, s1_ref, d_ref, ws_ref, wn_ref,
                b_ref, g_ref, be_ref, o_ref):
    neigh = s0_ref[0] + s1_ref[0]
    deg = d_ref[...]
    nm = neigh * (1.0 / jnp.maximum(deg, 1.0))
    y = (jnp.dot(h_ref[...], ws_ref[...], preferred_element_type=jnp.float32)
         + jnp.dot(nm, wn_ref[...], preferred_element_type=jnp.float32)
         + b_ref[...])
    mu = jnp.mean(y, axis=-1, keepdims=True)
    var = jnp.mean((y - mu) ** 2, axis=-1, keepdims=True)
    out = (y - mu) * lax.rsqrt(var + 1e-5) * g_ref[...] + be_ref[...]
    if relu:
        out = jnp.maximum(out, 0.0)
    o_ref[...] = out


def _dense_layer(h, s, dg, ws, wn, b, g, be, relu):
    return pl.pallas_call(
        functools.partial(_dense_body, relu),
        grid=(N // BN,),
        in_specs=[
            pl.BlockSpec((BN, D), lambda i: (i, 0)),
            pl.BlockSpec((1, BN, D), lambda i: (0, i, 0)),
            pl.BlockSpec((1, BN, D), lambda i: (1, i, 0)),
            pl.BlockSpec((BN, 1), lambda i: (i, 0)),
            pl.BlockSpec((D, D), lambda i: (0, 0)),
            pl.BlockSpec((D, D), lambda i: (0, 0)),
            pl.BlockSpec((1, D), lambda i: (0, 0)),
            pl.BlockSpec((1, D), lambda i: (0, 0)),
            pl.BlockSpec((1, D), lambda i: (0, 0)),
        ],
        out_specs=pl.BlockSpec((BN, D), lambda i: (i, 0)),
        out_shape=jax.ShapeDtypeStruct((N, D), jnp.float32),
    )(h, s, s, dg, ws, wn, b, g, be)


def _hp_body(xp_ref, wl_ref, bl_ref, g_ref, be_ref, o_ref):
    hp = xp_ref[...]
    for l in range(L):
        y = (jnp.dot(hp, wl_ref[l], preferred_element_type=jnp.float32)
             + bl_ref[l])
        mu = jnp.mean(y, axis=-1, keepdims=True)
        var = jnp.mean((y - mu) ** 2, axis=-1, keepdims=True)
        hp = (y - mu) * lax.rsqrt(var + 1e-5) * g_ref[l] + be_ref[l]
        if l < L - 1:
            hp = jnp.maximum(hp, 0.0)
    o_ref[...] = hp


def _hp_path(xp, wl, bl, g, be):
    return pl.pallas_call(
        _hp_body,
        out_shape=jax.ShapeDtypeStruct((NP, D), jnp.float32),
    )(xp, wl, bl, g, be)


def kernel(x, xp, edge_index, Ws, Wn, b, Wl, bl, gamma, beta):
    src = edge_index[0]
    dst1 = edge_index[1]
    dst2 = dst1.reshape(NW, EPW)
    dstm = dst2[:, :NCHUNK * C].reshape(NW, NCHUNK, C)
    dstt = dst2[:, NCHUNK * C:]
    b2 = b.reshape(L, 1, D)
    bl2 = bl.reshape(L, 1, D)
    g2 = gamma.reshape(L, 1, D)
    be2 = beta.reshape(L, 1, D)

    (dg3,) = _sc_deg(dst1)
    dg = (dg3[0, 0, :N] + dg3[1, 0, :N]).reshape(N, 1)

    h = x
    for l in range(L):
        (s,) = _sc_segsum(h, src, dstm, dstt)
        h = _dense_layer(h, s, dg, Ws[l], Wn[l], b2[l], g2[l], be2[l],
                         relu=(l < L - 1))
    hp = _hp_path(xp, Wl, bl2, g2, be2)
    return h, hp


# speedup vs baseline: 1.2354x; 1.0124x over previous
import functools

import jax
import jax.numpy as jnp
from jax import lax
from jax.experimental import pallas as pl
from jax.experimental.pallas import tpu as pltpu
from jax.experimental.pallas import tpu_sc as plsc

N = 10000
NP = 1024
E = 320000
D = 128
L = 3

NC = 2
NS = 16
NW = NC * NS
EPW = E // NW
C = 104
NCHUNK = EPW // C
TAIL = EPW - NCHUNK * C
ZR = 16
ZZ = 24
SLAB = (N // NS) // ZR * ZR
NPAD = 10240
SLABC = NPAD // NS


def _make_sc_segsum():
    mesh = plsc.VectorSubcoreMesh(core_axis_name="c", subcore_axis_name="s")

    out_type = [jax.ShapeDtypeStruct((NC, N, D), jnp.float32)]
    scratch = [
        pltpu.VMEM((NCHUNK, C), jnp.int32),
        pltpu.VMEM((TAIL,), jnp.int32),
        pltpu.VMEM((C, D), jnp.float32),
        pltpu.VMEM_SHARED((N, D), jnp.float32),
        pltpu.VMEM((EPW,), jnp.int32),
        pltpu.VMEM((C, D), jnp.float32),
        pltpu.SemaphoreType.DMA,
        pltpu.SemaphoreType.DMA,
        pltpu.SemaphoreType.DMA,
        pltpu.SemaphoreType.DMA,
    ]

    def body(*refs):
        (h_hbm, src_hbm, dstm_hbm, dstt_hbm, s_out, dst_v, dstt_v, rows_a,
         acc_sh, src_v, rows_b, sem_a, sem_b, sem_sa, sem_sb) = refs

        cid = lax.axis_index("c")
        sid = lax.axis_index("s")
        w = cid * NS + sid

        pltpu.sync_copy(dstm_hbm.at[w], dst_v)
        pltpu.sync_copy(dstt_hbm.at[w], dstt_v)
        pltpu.sync_copy(
            src_hbm.at[pl.ds(pl.multiple_of(w * EPW, 8), EPW)], src_v)

        def src_at(j, n=C):
            return src_v.at[pl.ds(pl.multiple_of(j * C, 8), n)]

        def zfill(i, _):
            rows_b[i // 8, pl.ds((i % 8) * 16, 16)] = jnp.zeros((16,), jnp.float32)
            return 0
        lax.fori_loop(0, C * 8, zfill, 0)

        for k in range(SLAB // C):
            r = sid * SLAB + k * C
            pltpu.sync_copy(rows_b, acc_sh.at[pl.ds(r, C)])
        pltpu.sync_copy(rows_b.at[pl.ds(0, ZR)], acc_sh.at[pl.ds(NS * SLAB, ZR)])
        plsc.subcore_barrier()

        def wait_g(buf, sem):
            pltpu.make_async_copy(h_hbm.at[dst_v.at[0]], buf, sem).wait()

        pltpu.async_copy(h_hbm.at[src_at(0)], rows_a, sem_a)

        def step(jj, _):
            b1 = 2 * jj + 1
            b2 = 2 * jj + 2
            gb = pltpu.async_copy(h_hbm.at[src_at(b1)], rows_b, sem_b)
            wait_g(rows_a, sem_a)
            sa = pltpu.async_copy(rows_a, acc_sh.at[dst_v.at[2 * jj]],
                                  sem_sa, add=True)
            sa.wait()
            pltpu.async_copy(h_hbm.at[src_at(b2)], rows_a, sem_a)
            gb.wait()
            sb = pltpu.async_copy(rows_b, acc_sh.at[dst_v.at[b1]],
                                  sem_sb, add=True)
            sb.wait()
            return 0
        lax.fori_loop(0, (NCHUNK - 2) // 2, step, 0)
        g95 = pltpu.async_copy(h_hbm.at[src_at(NCHUNK - 1)], rows_b, sem_b)
        wait_g(rows_a, sem_a)
        pltpu.sync_copy(rows_a, acc_sh.at[dst_v.at[NCHUNK - 2]], add=True)
        gt = pltpu.async_copy(h_hbm.at[src_at(NCHUNK, TAIL)],
                              rows_a.at[pl.ds(0, TAIL)], sem_a)
        g95.wait()
        pltpu.sync_copy(rows_b, acc_sh.at[dst_v.at[NCHUNK - 1]], add=True)
        gt.wait()
        pltpu.sync_copy(rows_a.at[pl.ds(0, TAIL)], acc_sh.at[dstt_v],
                        add=True)
        plsc.subcore_barrier()

        r0 = pl.multiple_of(sid * SLAB, 8)
        pltpu.sync_copy(acc_sh.at[pl.ds(r0, SLAB)],
                        s_out.at[cid, pl.ds(r0, SLAB)])
        pltpu.sync_copy(acc_sh.at[pl.ds(NS * SLAB, ZR)],
                        s_out.at[cid, pl.ds(NS * SLAB, ZR)])

    return pl.kernel(body, out_type=out_type, mesh=mesh,
                     scratch_types=scratch)


def _make_sc_deg():
    mesh = plsc.VectorSubcoreMesh(core_axis_name="c", subcore_axis_name="s")

    out_type = [jax.ShapeDtypeStruct((NC, 8, NPAD), jnp.float32)]
    scratch = [
        pltpu.VMEM((EPW,), jnp.int32),
        pltpu.VMEM((NPAD,), jnp.float32),
        pltpu.VMEM((NS, SLABC), jnp.float32),
        pltpu.VMEM((SLABC,), jnp.float32),
        pltpu.VMEM_SHARED((NS, NPAD), jnp.float32),
    ]

    def body(dst_hbm, out, dst_v, hist_v, slab_v, outv_v, hsh):
        cid = lax.axis_index("c")
        sid = lax.axis_index("s")
        w = cid * NS + sid
        pltpu.sync_copy(
            dst_hbm.at[pl.ds(pl.multiple_of(w * EPW, 8), EPW)], dst_v)

        zero16 = jnp.zeros((16,), jnp.float32)
        one16 = jnp.full((16,), 1.0, jnp.float32)

        def zh(i, _):
            hist_v[pl.ds(i * 16, 16)] = zero16
            return 0
        lax.fori_loop(0, NPAD // 16, zh, 0)

        def hchunk(i, _):
            idx = dst_v[pl.ds(i * 16, 16)]
            plsc.addupdate_scatter(hist_v, [idx], one16)
            return 0
        lax.fori_loop(0, EPW // 16, hchunk, 0)

        pltpu.sync_copy(hist_v, hsh.at[sid])
        plsc.subcore_barrier()

        c0 = pl.multiple_of(sid * SLABC, 128)
        pltpu.sync_copy(hsh.at[:, pl.ds(c0, SLABC)], slab_v)

        def red(q, _):
            acc = zero16
            for r in range(NS):
                acc = acc + slab_v[r, pl.ds(q * 16, 16)]
            outv_v[pl.ds(q * 16, 16)] = acc
            return 0
        lax.fori_loop(0, SLABC // 16, red, 0)

        pltpu.sync_copy(outv_v, out.at[cid, 0, pl.ds(c0, SLABC)])

    return pl.kernel(
        body, out_type=out_type, mesh=mesh, scratch_types=scratch,
        compiler_params=pltpu.CompilerParams(needs_layout_passes=False))


_sc_segsum = _make_sc_segsum()
_sc_deg = _make_sc_deg()


BN = 5000


def _dense_body(relu, h_ref, s0_ref, s1_ref, d_ref, ws_ref, wn_ref,
                b_ref, g_ref, be_ref, o_ref):
    neigh = s0_ref[0] + s1_ref[0]
    deg = d_ref[...]
    nm = neigh * (1.0 / jnp.maximum(deg, 1.0))
    y = (jnp.dot(h_ref[...], ws_ref[...], preferred_element_type=jnp.float32)
         + jnp.dot(nm, wn_ref[...], preferred_element_type=jnp.float32)
         + b_ref[...])
    mu = jnp.mean(y, axis=-1, keepdims=True)
    var = jnp.mean((y - mu) ** 2, axis=-1, keepdims=True)
    out = (y - mu) * lax.rsqrt(var + 1e-5) * g_ref[...] + be_ref[...]
    if relu:
        out = jnp.maximum(out, 0.0)
    o_ref[...] = out


def _dense_layer(h, s, dg, ws, wn, b, g, be, relu):
    return pl.pallas_call(
        functools.partial(_dense_body, relu),
        grid=(N // BN,),
        in_specs=[
            pl.BlockSpec((BN, D), lambda i: (i, 0)),
            pl.BlockSpec((1, BN, D), lambda i: (0, i, 0)),
            pl.BlockSpec((1, BN, D), lambda i: (1, i, 0)),
            pl.BlockSpec((BN, 1), lambda i: (i, 0)),
            pl.BlockSpec((D, D), lambda i: (0, 0)),
            pl.BlockSpec((D, D), lambda i: (0, 0)),
            pl.BlockSpec((1, D), lambda i: (0, 0)),
            pl.BlockSpec((1, D), lambda i: (0, 0)),
            pl.BlockSpec((1, D), lambda i: (0, 0)),
        ],
        out_specs=pl.BlockSpec((BN, D), lambda i: (i, 0)),
        out_shape=jax.ShapeDtypeStruct((N, D), jnp.float32),
    )(h, s, s, dg, ws, wn, b, g, be)


def _hp_body(xp_ref, wl_ref, bl_ref, g_ref, be_ref, o_ref):
    hp = xp_ref[...]
    for l in range(L):
        y = (jnp.dot(hp, wl_ref[l], preferred_element_type=jnp.float32)
             + bl_ref[l])
        mu = jnp.mean(y, axis=-1, keepdims=True)
        var = jnp.mean((y - mu) ** 2, axis=-1, keepdims=True)
        hp = (y - mu) * lax.rsqrt(var + 1e-5) * g_ref[l] + be_ref[l]
        if l < L - 1:
            hp = jnp.maximum(hp, 0.0)
    o_ref[...] = hp


def _hp_path(xp, wl, bl, g, be):
    return pl.pallas_call(
        _hp_body,
        out_shape=jax.ShapeDtypeStruct((NP, D), jnp.float32),
    )(xp, wl, bl, g, be)


def kernel(x, xp, edge_index, Ws, Wn, b, Wl, bl, gamma, beta):
    src = edge_index[0]
    dst1 = edge_index[1]
    dst2 = dst1.reshape(NW, EPW)
    dstm = dst2[:, :NCHUNK * C].reshape(NW, NCHUNK, C)
    dstt = dst2[:, NCHUNK * C:]
    b2 = b.reshape(L, 1, D)
    bl2 = bl.reshape(L, 1, D)
    g2 = gamma.reshape(L, 1, D)
    be2 = beta.reshape(L, 1, D)

    (dg3,) = _sc_deg(dst1)
    dg = (dg3[0, 0, :N] + dg3[1, 0, :N]).reshape(N, 1)

    h = x
    for l in range(L):
        (s,) = _sc_segsum(h, src, dstm, dstt)
        h = _dense_layer(h, s, dg, Ws[l], Wn[l], b2[l], g2[l], be2[l],
                         relu=(l < L - 1))
    hp = _hp_path(xp, Wl, bl2, g2, be2)
    return h, hp
